# baseline (speedup 1.0000x reference)
import functools

import jax
import jax.numpy as jnp
from jax import lax
from jax.experimental import pallas as pl
from jax.experimental.pallas import tpu as pltpu
from jax.experimental.pallas import tpu_sc as plsc

NC = 2
NS = 16
L = 16
NW = NC * NS
B = 16384
D = 32
BPW = B // NW
NJ = BPW // 128


def _sc_diff(user_t, item_t, user_ids, pos_ids, neg_ids):
    mesh = plsc.VectorSubcoreMesh(core_axis_name="c", subcore_axis_name="s")

    @functools.partial(
        pl.kernel,
        mesh=mesh,
        compiler_params=pltpu.CompilerParams(
            needs_layout_passes=False, use_tc_tiling_on_sc=False
        ),
        out_type=jax.ShapeDtypeStruct((B,), jnp.float32),
        scratch_types=[
            pltpu.VMEM((NJ, 128), jnp.int32),
            pltpu.VMEM((NJ, 128), jnp.int32),
            pltpu.VMEM((NJ, 128), jnp.int32),
            pltpu.VMEM((2 * BPW,), jnp.float32),
            pltpu.VMEM((2 * BPW,), jnp.float32),
            pltpu.VMEM((2 * BPW,), jnp.float32),
            pltpu.VMEM((BPW,), jnp.float32),
            pltpu.SemaphoreType.DMA((2,)),
            pltpu.SemaphoreType.DMA((2,)),
            pltpu.SemaphoreType.DMA((2,)),
            pltpu.SemaphoreType.DMA,
        ],
    )
    def k(ut, it, uids, pids, nids, out, iu, ip, inn, ub, pb, nb, acc,
          usem, psem, nsem, isem):
        wid = lax.axis_index("s") * NC + lax.axis_index("c")
        base = wid * BPW
        for j in range(NJ):
            sl = pl.ds(base + j * 128, 128)
            pltpu.async_copy(uids.at[sl], iu.at[j], isem)
            pltpu.async_copy(pids.at[sl], ip.at[j], isem)
            pltpu.async_copy(nids.at[sl], inn.at[j], isem)
        for j in range(NJ):
            sl = pl.ds(base + j * 128, 128)
            pltpu.make_async_copy(uids.at[sl], iu.at[j], isem).wait()
            pltpu.make_async_copy(pids.at[sl], ip.at[j], isem).wait()
            pltpu.make_async_copy(nids.at[sl], inn.at[j], isem).wait()
        for c in range(BPW // L):
            acc[pl.ds(c * L, L)] = jnp.zeros((L,), jnp.float32)

        def fire(d, b):
            off = b * BPW
            for j in range(NJ):
                dsl = pl.ds(off + j * 128, 128)
                pltpu.async_copy(ut.at[d].at[iu.at[j]], ub.at[dsl], usem.at[b])
                pltpu.async_copy(it.at[d].at[ip.at[j]], pb.at[dsl], psem.at[b])
                pltpu.async_copy(it.at[d].at[inn.at[j]], nb.at[dsl], nsem.at[b])

        def drain(d, b):
            off = b * BPW
            for j in range(NJ):
                dsl = pl.ds(off + j * 128, 128)
                pltpu.make_async_copy(
                    ut.at[d].at[iu.at[j]], ub.at[dsl], usem.at[b]).wait()
                pltpu.make_async_copy(
                    it.at[d].at[ip.at[j]], pb.at[dsl], psem.at[b]).wait()
                pltpu.make_async_copy(
                    it.at[d].at[inn.at[j]], nb.at[dsl], nsem.at[b]).wait()

        fire(0, 0)

        def body(d, carry):
            b = lax.rem(d, 2)

            @pl.when(d < D - 1)
            def _():
                fire(d + 1, 1 - b)

            drain(d, b)
            off = b * BPW
            for c in range(BPW // L):
                csl = pl.ds(off + c * L, L)
                asl = pl.ds(c * L, L)
                acc[asl] = acc[asl] + ub[csl] * (pb[csl] - nb[csl])
            return carry

        lax.fori_loop(0, D, body, 0)
        pltpu.sync_copy(acc, out.at[pl.ds(base, BPW)])

    return k(user_t, item_t, user_ids, pos_ids, neg_ids)


def _tc_loss_kernel(x_ref, o_ref):
    o_ref[0, 0] = -jnp.sum(jax.nn.log_sigmoid(x_ref[:, :]))


def _tc_loss(diff):
    x = diff.reshape(B // 128, 128)
    res = pl.pallas_call(
        _tc_loss_kernel,
        out_shape=jax.ShapeDtypeStruct((1, 1), jnp.float32),
        out_specs=pl.BlockSpec(memory_space=pltpu.SMEM),
    )(x)
    return res[0, 0]


def kernel(user_emb, item_emb, user_ids, pos_ids, neg_ids):
    n_users = user_emb.shape[0]
    n_items = item_emb.shape[0]
    user_t = user_emb.T
    item_t = item_emb.T
    diff = _sc_diff(user_t, item_t, user_ids, pos_ids, neg_ids)
    return _tc_loss(diff)


# speedup vs baseline: 11.6687x; 11.6687x over previous
import functools

import jax
import jax.numpy as jnp
from jax import lax
from jax.experimental import pallas as pl
from jax.experimental.pallas import tpu as pltpu
from jax.experimental.pallas import tpu_sc as plsc

NC = 2
NS = 16
L = 16
NW = NC * NS
B = 16384
D = 32
BPW = B // NW
NG = BPW // L


def _sc_diff(user_t, item_t, user_ids, pos_ids, neg_ids):
    mesh = plsc.VectorSubcoreMesh(core_axis_name="c", subcore_axis_name="s")

    @functools.partial(
        pl.kernel,
        mesh=mesh,
        compiler_params=pltpu.CompilerParams(
            needs_layout_passes=False, use_tc_tiling_on_sc=True
        ),
        out_type=jax.ShapeDtypeStruct((B,), jnp.float32),
        scratch_types=[
            pltpu.VMEM((BPW,), jnp.int32),
            pltpu.VMEM((BPW,), jnp.int32),
            pltpu.VMEM((BPW,), jnp.int32),
            pltpu.VMEM((2, 4, 8, 128), jnp.float32),
            pltpu.VMEM((2, 4, 8, 128), jnp.float32),
            pltpu.VMEM((2, 4, 8, 128), jnp.float32),
            pltpu.VMEM((BPW,), jnp.float32),
            pltpu.SemaphoreType.DMA((2,)),
            pltpu.SemaphoreType.DMA((2,)),
            pltpu.SemaphoreType.DMA((2,)),
            pltpu.SemaphoreType.DMA,
        ],
    )
    def k(ut, it, uids, pids, nids, out, iu, ip, inn, gu, gp, gn, dv,
          usem, psem, nsem, isem):
        wid = lax.axis_index("s") * NC + lax.axis_index("c")
        base = wid * BPW
        sl = pl.ds(base, BPW)
        pltpu.async_copy(uids.at[sl], iu, isem)
        pltpu.async_copy(pids.at[sl], ip, isem)
        pltpu.async_copy(nids.at[sl], inn, isem)
        pltpu.make_async_copy(uids.at[sl], iu, isem).wait()
        pltpu.make_async_copy(pids.at[sl], ip, isem).wait()
        pltpu.make_async_copy(nids.at[sl], inn, isem).wait()

        def fire(g, b):
            cu = iu[pl.ds(g * L, L)]
            cp = ip[pl.ds(g * L, L)]
            cn = inn[pl.ds(g * L, L)]
            for i in range(L):
                dst = pl.ds(i * 8, 8)
                r0 = (cu[i] // 8) * 8
                pltpu.async_copy(
                    ut.at[:, :, pl.ds(r0, 8)],
                    gu.at[b, :, :, dst], usem.at[b])
                r0 = (cp[i] // 8) * 8
                pltpu.async_copy(
                    it.at[:, :, pl.ds(r0, 8)],
                    gp.at[b, :, :, dst], psem.at[b])
                r0 = (cn[i] // 8) * 8
                pltpu.async_copy(
                    it.at[:, :, pl.ds(r0, 8)],
                    gn.at[b, :, :, dst], nsem.at[b])

        fire(0, 0)
        iota = lax.iota(jnp.int32, L)

        def body(g, carry):
            b = lax.rem(g, 2)

            @pl.when(g < NG - 1)
            def _():
                fire(g + 1, 1 - b)

            dummy = ut.at[:, :, pl.ds(0, 128)]
            pltpu.make_async_copy(dummy, gu.at[b], usem.at[b]).wait()
            pltpu.make_async_copy(dummy, gp.at[b], psem.at[b]).wait()
            pltpu.make_async_copy(dummy, gn.at[b], nsem.at[b]).wait()

            bv = jnp.full((L,), b, jnp.int32)
            pos_u = iota * 8 + (iu[pl.ds(g * L, L)] & 7)
            pos_p = iota * 8 + (ip[pl.ds(g * L, L)] & 7)
            pos_n = iota * 8 + (inn[pl.ds(g * L, L)] & 7)
            acc = jnp.zeros((L,), jnp.float32)
            for d in range(D):
                dtv = jnp.full((L,), d // 8, jnp.int32)
                sv = jnp.full((L,), d % 8, jnp.int32)
                uv = plsc.load_gather(gu, [bv, dtv, sv, pos_u])
                pv = plsc.load_gather(gp, [bv, dtv, sv, pos_p])
                nv = plsc.load_gather(gn, [bv, dtv, sv, pos_n])
                acc = acc + uv * (pv - nv)
            dv[pl.ds(g * L, L)] = acc
            return carry

        lax.fori_loop(0, NG, body, 0)
        pltpu.sync_copy(dv, out.at[sl])

    return k(user_t, item_t, user_ids, pos_ids, neg_ids)


def _tc_loss_kernel(x_ref, o_ref):
    o_ref[0, 0] = -jnp.sum(jax.nn.log_sigmoid(x_ref[:, :]))


def _tc_loss(diff):
    x = diff.reshape(B // 128, 128)
    res = pl.pallas_call(
        _tc_loss_kernel,
        out_shape=jax.ShapeDtypeStruct((1, 1), jnp.float32),
        out_specs=pl.BlockSpec(memory_space=pltpu.SMEM),
    )(x)
    return res[0, 0]


def kernel(user_emb, item_emb, user_ids, pos_ids, neg_ids):
    n_users = user_emb.shape[0]
    n_items = item_emb.shape[0]
    user_t = user_emb.T.reshape(D // 8, 8, n_users)
    item_t = item_emb.T.reshape(D // 8, 8, n_items)
    diff = _sc_diff(user_t, item_t, user_ids, pos_ids, neg_ids)
    return _tc_loss(diff)
